# Initial kernel scaffold; baseline (speedup 1.0000x reference)
import jax
import jax.numpy as jnp
from jax.experimental import pallas as pl


def kernel(input, weight):
    raise NotImplementedError("write your pallas kernel here")


# speedup vs baseline: 1.8421x; 1.8421x over previous
import functools

import jax
import jax.numpy as jnp
from jax import lax
from jax.experimental import pallas as pl
from jax.experimental.pallas import tpu as pltpu
from jax.experimental.pallas import tpu_sc as plsc

DIM = 64
NC = 2
NS = 16
NW = NC * NS

CHUNK = 128
K = 8


@functools.partial(jax.jit, static_argnames=("batch_rows",))
def _sc_gather(idx2d, weight, batch_rows):
    b_per_w = batch_rows // NW
    slab = K * CHUNK
    num_slabs = b_per_w // slab
    mesh = plsc.VectorSubcoreMesh(core_axis_name="c", subcore_axis_name="s")

    @functools.partial(
        pl.kernel,
        mesh=mesh,
        out_type=jax.ShapeDtypeStruct((batch_rows, DIM), jnp.float32),
        scratch_types=[
            pltpu.VMEM((K, CHUNK), jnp.int32),
            pltpu.VMEM((slab, DIM), jnp.float32),
            pltpu.SemaphoreType.DMA,
        ],
        compiler_params=pltpu.CompilerParams(use_tc_tiling_on_sc=False),
    )
    def body(idx_hbm, table_hbm, out_hbm, idx_v, rows_v, sem):
        wid = lax.axis_index("s") * NC + lax.axis_index("c")
        base = wid * b_per_w

        def step(s, carry):
            off = pl.multiple_of(base + s * slab, slab)
            pltpu.sync_copy(
                idx_hbm.at[pl.ds(pl.multiple_of(off // CHUNK, K), K)], idx_v)
            copies = []
            for j in range(K):
                copies.append(pltpu.async_copy(
                    table_hbm.at[idx_v.at[j]],
                    rows_v.at[pl.ds(j * CHUNK, CHUNK)],
                    sem,
                ))
            for c in copies:
                c.wait()
            pltpu.sync_copy(rows_v, out_hbm.at[pl.ds(off, slab)])
            return carry

        lax.fori_loop(0, num_slabs, step, 0)

    return body(idx2d, weight)


def kernel(input, weight):
    b, h = input.shape
    batch_rows = b * h
    idx2d = input.reshape(batch_rows // CHUNK, CHUNK).astype(jnp.int32)
    out = _sc_gather(idx2d, weight, batch_rows)
    return out.reshape(b, h, DIM)


# speedup vs baseline: 1.8545x; 1.0067x over previous
import functools

import jax
import jax.numpy as jnp
from jax import lax
from jax.experimental import pallas as pl
from jax.experimental.pallas import tpu as pltpu
from jax.experimental.pallas import tpu_sc as plsc

DIM = 64
NC = 2
NS = 16
NW = NC * NS

CHUNK = 128
K = 4
SLAB = K * CHUNK


@functools.partial(jax.jit, static_argnames=("batch_rows",))
def _sc_gather(idx3d, weight, batch_rows):
    b_per_w = batch_rows // NW
    num_slabs = b_per_w // SLAB
    mesh = plsc.VectorSubcoreMesh(core_axis_name="c", subcore_axis_name="s")

    @functools.partial(
        pl.kernel,
        mesh=mesh,
        out_type=jax.ShapeDtypeStruct((batch_rows // SLAB, SLAB, DIM),
                                      jnp.float32),
        scratch_types=[
            pltpu.VMEM((2, K, CHUNK), jnp.int32),
            pltpu.VMEM((2, SLAB, DIM), jnp.float32),
            pltpu.SemaphoreType.DMA,
            pltpu.SemaphoreType.DMA,
        ],
        compiler_params=pltpu.CompilerParams(use_tc_tiling_on_sc=False),
    )
    def body(idx_hbm, table_hbm, out_hbm, idx_v, rows_v, sem0, sem1):
        wid = lax.axis_index("s") * NC + lax.axis_index("c")
        slab0 = wid * num_slabs
        sems = (sem0, sem1)

        def fire(s, p):
            pltpu.sync_copy(idx_hbm.at[slab0 + s], idx_v.at[p])
            for j in range(K):
                pltpu.async_copy(
                    table_hbm.at[idx_v.at[p].at[j]],
                    rows_v.at[p].at[pl.ds(j * CHUNK, CHUNK)],
                    sems[p],
                )

        def drain_store(s, p):
            for j in range(K):
                pltpu.make_async_copy(
                    table_hbm.at[pl.ds(0, CHUNK)],
                    rows_v.at[p].at[pl.ds(j * CHUNK, CHUNK)],
                    sems[p],
                ).wait()
            pltpu.sync_copy(rows_v.at[p], out_hbm.at[slab0 + s])

        fire(0, 0)

        def step2(s2, carry):
            for p in range(2):
                s = s2 * 2 + p

                @pl.when(s + 1 < num_slabs)
                def _():
                    fire(s + 1, (p + 1) % 2)

                drain_store(s, p)
            return carry

        lax.fori_loop(0, num_slabs // 2, step2, 0)

    return body(idx3d, weight)


def kernel(input, weight):
    b, h = input.shape
    batch_rows = b * h
    idx3d = input.reshape(batch_rows // SLAB, K, CHUNK).astype(jnp.int32)
    out = _sc_gather(idx3d, weight, batch_rows)
    return out.reshape(b, h, DIM)
